# Initial kernel scaffold; baseline (speedup 1.0000x reference)
import jax
import jax.numpy as jnp
from jax.experimental import pallas as pl


def kernel(element, fraction, element_indices, Ws1_w, Ws1_b, Ws2_w, Ws2_b, Wc1_w, Wc1_b, Wc2_w, Wc2_b):
    raise NotImplementedError("write your pallas kernel here")


# speedup vs baseline: 8.0967x; 8.0967x over previous
import jax
import jax.numpy as jnp
from jax import lax
from jax.experimental import pallas as pl

_B, _L, _D, _H, _HID = 256, 8, 128, 2, 256
_BT = 32
_N = _BT * _L * _L
_R = _BT * _L


def _leaky(x):
    return jnp.maximum(x, 0.01 * x)


def _attn_kernel(el_ref, fr_ref, gie_ref, gio_ref,
                 ws1_ref, ws1b_ref, ws2_ref, ws2b_ref,
                 wc1_ref, wc1b_ref, wc2_ref, wc2b_ref,
                 out_ref):
    el = el_ref[...]
    el2 = el.reshape(_R, _D)
    fr = fr_ref[...]
    gie = gie_ref[...]
    gio = gio_ref[...]

    base = pl.program_id(0) * _R
    col = lax.broadcasted_iota(jnp.int32, (_N, _R), 1) + base
    ohe = (gie == col).astype(jnp.float32)
    oho = (gio == col).astype(jnp.float32)
    ae = jnp.dot(ohe, el2, preferred_element_type=jnp.float32)
    ao = jnp.dot(oho, el2, preferred_element_type=jnp.float32)

    acc = jnp.zeros((_R, _D), jnp.float32)
    for h in range(_H):
        w1 = ws1_ref[h]
        h1 = (jnp.dot(ae, w1[:_D, :], preferred_element_type=jnp.float32)
              + jnp.dot(ao, w1[_D:, :], preferred_element_type=jnp.float32)
              + ws1b_ref[h][None, :])
        h1 = _leaky(h1)
        aw = (jnp.sum(h1 * ws2_ref[h][:, 0][None, :], axis=1, keepdims=True)
              + ws2b_ref[h][None, :])
        w = jnp.exp(aw) * fr

        wc1 = wc1_ref[h]
        c1 = (jnp.dot(ae, wc1[:_D, :], preferred_element_type=jnp.float32)
              + jnp.dot(ao, wc1[_D:, :], preferred_element_type=jnp.float32)
              + wc1b_ref[h][None, :])
        c1 = _leaky(c1)
        c = (jnp.dot(c1, wc2_ref[h], preferred_element_type=jnp.float32)
             + wc2b_ref[h][None, :])

        cw = c * w
        wl = w * jnp.ones((1, _D), jnp.float32)
        num = jnp.sum(cw.reshape(_R, _L, _D), axis=1)
        den = jnp.sum(wl.reshape(_R, _L, _D), axis=1)
        acc = acc + num / den

    res = el2 + acc * (1.0 / _H)
    out_ref[...] = res.reshape(_BT, _L, _D)


def kernel(element, fraction, element_indices, Ws1_w, Ws1_b, Ws2_w, Ws2_b,
           Wc1_w, Wc1_b, Wc2_w, Wc2_b):
    b = element.shape[0]
    nrows = b * _L * _L
    idx = element_indices.reshape(b, _L * _L, 2).astype(jnp.int32)
    boff = (_L * jnp.arange(b, dtype=jnp.int32))[:, None]
    gie = (idx[..., 0] + boff).reshape(nrows, 1)
    gio = (idx[..., 1] + boff).reshape(nrows, 1)
    frn = jnp.tile(fraction, (1, _L)).reshape(nrows, 1)

    grid = (b // _BT,)
    full = lambda a: pl.BlockSpec(a.shape, lambda i: (0,) * a.ndim)
    out = pl.pallas_call(
        _attn_kernel,
        grid=grid,
        in_specs=[
            pl.BlockSpec((_BT, _L, _D), lambda i: (i, 0, 0)),
            pl.BlockSpec((_N, 1), lambda i: (i, 0)),
            pl.BlockSpec((_N, 1), lambda i: (i, 0)),
            pl.BlockSpec((_N, 1), lambda i: (i, 0)),
            full(Ws1_w), full(Ws1_b), full(Ws2_w), full(Ws2_b),
            full(Wc1_w), full(Wc1_b), full(Wc2_w), full(Wc2_b),
        ],
        out_specs=pl.BlockSpec((_BT, _L, _D), lambda i: (i, 0, 0)),
        out_shape=jax.ShapeDtypeStruct((b, _L, _D), jnp.float32),
    )(element, frn, gie, gio, Ws1_w, Ws1_b, Ws2_w, Ws2_b,
      Wc1_w, Wc1_b, Wc2_w, Wc2_b)
    return out


# speedup vs baseline: 8.4183x; 1.0397x over previous
import jax
import jax.numpy as jnp
from jax import lax
from jax.experimental import pallas as pl

_B, _L, _D, _H, _HID = 256, 8, 128, 2, 256
_BT = 32
_N = _BT * _L * _L
_R = _BT * _L


def _leaky(x):
    return jnp.maximum(x, 0.01 * x)


def _attn_kernel(el_ref, fr_ref, gie_ref, gio_ref,
                 w1t_ref, w1b_ref, b1_ref, ws2_ref, ws2b_ref,
                 wc2_ref, wc2b_ref, out_ref):
    el = el_ref[...]
    el2 = el.reshape(_R, _D)
    fr = fr_ref[...]
    gie = gie_ref[...]
    gio = gio_ref[...]

    base = pl.program_id(0) * _R
    col = lax.broadcasted_iota(jnp.int32, (_N, _R), 1) + base
    ohe = (gie == col).astype(jnp.bfloat16)
    oho = (gio == col).astype(jnp.bfloat16)
    el2b = el2.astype(jnp.bfloat16)
    ae = jnp.dot(ohe, el2b, preferred_element_type=jnp.float32)
    ao = jnp.dot(oho, el2b, preferred_element_type=jnp.float32)
    aeb = ae.astype(jnp.bfloat16)
    aob = ao.astype(jnp.bfloat16)

    hall = (jnp.dot(aeb, w1t_ref[...], preferred_element_type=jnp.float32)
            + jnp.dot(aob, w1b_ref[...], preferred_element_type=jnp.float32)
            + b1_ref[...])
    hall = _leaky(hall)

    acc = jnp.zeros((_R, _D), jnp.float32)
    for h in range(_H):
        h1 = hall[:, h * _HID:(h + 1) * _HID]
        aw = (jnp.sum(h1 * ws2_ref[h][:, 0][None, :], axis=1, keepdims=True)
              + ws2b_ref[h][None, :])
        w = jnp.exp(aw) * fr

        c1 = hall[:, (2 + h) * _HID:(3 + h) * _HID].astype(jnp.bfloat16)
        c = (jnp.dot(c1, wc2_ref[h], preferred_element_type=jnp.float32)
             + wc2b_ref[h][None, :])

        cw = c * w
        wl = w * jnp.ones((1, _D), jnp.float32)
        num = jnp.sum(cw.reshape(_R, _L, _D), axis=1)
        den = jnp.sum(wl.reshape(_R, _L, _D), axis=1)
        acc = acc + num / den

    res = el2 + acc * (1.0 / _H)
    out_ref[...] = res.reshape(_BT, _L, _D)


def kernel(element, fraction, element_indices, Ws1_w, Ws1_b, Ws2_w, Ws2_b,
           Wc1_w, Wc1_b, Wc2_w, Wc2_b):
    b = element.shape[0]
    nrows = b * _L * _L
    idx = element_indices.reshape(b, _L * _L, 2).astype(jnp.int32)
    boff = (_L * jnp.arange(b, dtype=jnp.int32))[:, None]
    gie = (idx[..., 0] + boff).reshape(nrows, 1)
    gio = (idx[..., 1] + boff).reshape(nrows, 1)
    frn = jnp.tile(fraction, (1, _L)).reshape(nrows, 1)

    w1 = jnp.concatenate([Ws1_w[0], Ws1_w[1], Wc1_w[0], Wc1_w[1]], axis=1)
    w1t = w1[:_D].astype(jnp.bfloat16)
    w1b = w1[_D:].astype(jnp.bfloat16)
    b1 = jnp.concatenate([Ws1_b[0], Ws1_b[1], Wc1_b[0], Wc1_b[1]])[None, :]
    wc2b = Wc2_w.astype(jnp.bfloat16)

    grid = (b // _BT,)
    full = lambda a: pl.BlockSpec(a.shape, lambda i: (0,) * a.ndim)
    out = pl.pallas_call(
        _attn_kernel,
        grid=grid,
        in_specs=[
            pl.BlockSpec((_BT, _L, _D), lambda i: (i, 0, 0)),
            pl.BlockSpec((_N, 1), lambda i: (i, 0)),
            pl.BlockSpec((_N, 1), lambda i: (i, 0)),
            pl.BlockSpec((_N, 1), lambda i: (i, 0)),
            full(w1t), full(w1b), full(b1), full(Ws2_w), full(Ws2_b),
            full(wc2b), full(Wc2_b),
        ],
        out_specs=pl.BlockSpec((_BT, _L, _D), lambda i: (i, 0, 0)),
        out_shape=jax.ShapeDtypeStruct((b, _L, _D), jnp.float32),
    )(element, frn, gie, gio, w1t, w1b, b1, Ws2_w, Ws2_b, wc2b, Wc2_b)
    return out


# speedup vs baseline: 9.8868x; 1.1744x over previous
import jax
import jax.numpy as jnp
from jax import lax
from jax.experimental import pallas as pl

_B, _L, _D, _H, _HID = 256, 8, 128, 2, 256
_BT = 32
_N = _BT * _L * _L
_R = _BT * _L


def _leaky(x):
    return jnp.maximum(x, 0.01 * x)


def _attn_kernel(el_ref, fr_ref, gie_ref, gio_ref,
                 w1t_ref, w1b_ref, b1_ref, w2blk_ref, ws2b_ref,
                 wc2_ref, wc2b_ref, out_ref):
    el = el_ref[...]
    el2 = el.reshape(_R, _D)
    fr = fr_ref[...]
    gie = gie_ref[...]
    gio = gio_ref[...]

    base = pl.program_id(0) * _R
    col = lax.broadcasted_iota(jnp.int32, (_N, _R), 1) + base
    ohe = (gie == col).astype(jnp.bfloat16)
    oho = (gio == col).astype(jnp.bfloat16)
    el2b = el2.astype(jnp.bfloat16)
    ae = jnp.dot(ohe, el2b, preferred_element_type=jnp.float32)
    ao = jnp.dot(oho, el2b, preferred_element_type=jnp.float32)
    aeb = ae.astype(jnp.bfloat16)
    aob = ao.astype(jnp.bfloat16)

    hall = (jnp.dot(aeb, w1t_ref[...], preferred_element_type=jnp.float32)
            + jnp.dot(aob, w1b_ref[...], preferred_element_type=jnp.float32)
            + b1_ref[...])
    hall = _leaky(hall).astype(jnp.bfloat16)

    aw2 = jnp.dot(hall[:, :2 * _HID], w2blk_ref[...],
                  preferred_element_type=jnp.float32)

    acc = jnp.zeros((_R, _D), jnp.float32)
    for h in range(_H):
        aw = aw2[:, h:h + 1] + ws2b_ref[h][None, :]
        w = jnp.exp(aw) * fr

        c1 = hall[:, (2 + h) * _HID:(3 + h) * _HID]
        c = (jnp.dot(c1, wc2_ref[h], preferred_element_type=jnp.float32)
             + wc2b_ref[h][None, :])

        cw = c * w
        wl = w * jnp.ones((1, _D), jnp.float32)
        num = jnp.sum(cw.reshape(_R, _L, _D), axis=1)
        den = jnp.sum(wl.reshape(_R, _L, _D), axis=1)
        acc = acc + num / den

    res = el2 + acc * (1.0 / _H)
    out_ref[...] = res.reshape(_BT, _L, _D)


def kernel(element, fraction, element_indices, Ws1_w, Ws1_b, Ws2_w, Ws2_b,
           Wc1_w, Wc1_b, Wc2_w, Wc2_b):
    b = element.shape[0]
    nrows = b * _L * _L
    idx = element_indices.reshape(b, _L * _L, 2).astype(jnp.int32)
    boff = (_L * jnp.arange(b, dtype=jnp.int32))[:, None]
    gie = (idx[..., 0] + boff).reshape(nrows, 1)
    gio = (idx[..., 1] + boff).reshape(nrows, 1)
    frn = jnp.tile(fraction, (1, _L)).reshape(nrows, 1)

    w1 = jnp.concatenate([Ws1_w[0], Ws1_w[1], Wc1_w[0], Wc1_w[1]], axis=1)
    w1t = w1[:_D].astype(jnp.bfloat16)
    w1b = w1[_D:].astype(jnp.bfloat16)
    b1 = jnp.concatenate([Ws1_b[0], Ws1_b[1], Wc1_b[0], Wc1_b[1]])[None, :]
    wc2b = Wc2_w.astype(jnp.bfloat16)
    w2blk = jnp.zeros((2 * _HID, 128), jnp.float32)
    w2blk = w2blk.at[:_HID, 0].set(Ws2_w[0, :, 0]).at[_HID:, 1].set(Ws2_w[1, :, 0])
    w2blk = w2blk.astype(jnp.bfloat16)

    grid = (b // _BT,)
    full = lambda a: pl.BlockSpec(a.shape, lambda i: (0,) * a.ndim)
    out = pl.pallas_call(
        _attn_kernel,
        grid=grid,
        in_specs=[
            pl.BlockSpec((_BT, _L, _D), lambda i: (i, 0, 0)),
            pl.BlockSpec((_N, 1), lambda i: (i, 0)),
            pl.BlockSpec((_N, 1), lambda i: (i, 0)),
            pl.BlockSpec((_N, 1), lambda i: (i, 0)),
            full(w1t), full(w1b), full(b1), full(w2blk), full(Ws2_b),
            full(wc2b), full(Wc2_b),
        ],
        out_specs=pl.BlockSpec((_BT, _L, _D), lambda i: (i, 0, 0)),
        out_shape=jax.ShapeDtypeStruct((b, _L, _D), jnp.float32),
    )(element, frn, gie, gio, w1t, w1b, b1, w2blk, Ws2_b, wc2b, Wc2_b)
    return out


# speedup vs baseline: 10.7715x; 1.0895x over previous
import jax
import jax.numpy as jnp
from jax import lax
from jax.experimental import pallas as pl

_B, _L, _D, _H, _HID = 256, 8, 128, 2, 256
_BT = 32
_N = _BT * _L * _L
_R = _BT * _L


def _leaky(x):
    return jnp.maximum(x, 0.01 * x)


def _attn_kernel(el_ref, fr_ref, gb_ref,
                 w1c_ref, ws1b_ref, ws2_ref, ws2b_ref,
                 wc1b_ref, wc2_ref, wc2b_ref, out_ref):
    el = el_ref[...]
    el2 = el.reshape(_R, _D)
    fr = fr_ref[...]
    gb = gb_ref[...]

    base = pl.program_id(0) * _R
    col = lax.broadcasted_iota(jnp.int32, (_N, _R), 1) + base
    ohe = (gb[:, 0:1] == col).astype(jnp.bfloat16)
    oho = (gb[:, 1:2] == col).astype(jnp.bfloat16)
    el2b = el2.astype(jnp.bfloat16)
    ae = jnp.dot(ohe, el2b, preferred_element_type=jnp.float32)
    ao = jnp.dot(oho, el2b, preferred_element_type=jnp.float32)
    aeb = ae.astype(jnp.bfloat16)
    aob = ao.astype(jnp.bfloat16)

    halp = (jnp.dot(aeb, w1c_ref[:_D, :], preferred_element_type=jnp.float32)
            + jnp.dot(aob, w1c_ref[_D:, :], preferred_element_type=jnp.float32))

    acc = jnp.zeros((_R, _D), jnp.float32)
    for h in range(_H):
        h1 = _leaky(halp[:, h * _HID:(h + 1) * _HID]
                    + ws1b_ref[h][None, :]).astype(jnp.bfloat16)
        aw = (jnp.dot(h1, ws2_ref[h].astype(jnp.bfloat16),
                      preferred_element_type=jnp.float32)
              + ws2b_ref[h][None, :])
        w = jnp.exp(aw) * fr

        c1 = _leaky(halp[:, (2 + h) * _HID:(3 + h) * _HID]
                    + wc1b_ref[h][None, :]).astype(jnp.bfloat16)
        c = (jnp.dot(c1, wc2_ref[h].astype(jnp.bfloat16),
                     preferred_element_type=jnp.float32)
             + wc2b_ref[h][None, :])

        cw = c * w
        wl = w * jnp.ones((1, _D), jnp.float32)
        num = jnp.sum(cw.reshape(_R, _L, _D), axis=1)
        den = jnp.sum(wl.reshape(_R, _L, _D), axis=1)
        acc = acc + num / den

    res = el2 + acc * (1.0 / _H)
    out_ref[...] = res.reshape(_BT, _L, _D)


def kernel(element, fraction, element_indices, Ws1_w, Ws1_b, Ws2_w, Ws2_b,
           Wc1_w, Wc1_b, Wc2_w, Wc2_b):
    b = element.shape[0]
    nrows = b * _L * _L
    gboth = (element_indices.astype(jnp.int32)
             + (_L * jnp.arange(b, dtype=jnp.int32))[:, None]).reshape(nrows, 2)
    frn = jnp.tile(fraction, (1, _L)).reshape(nrows, 1)
    w1c = jnp.concatenate([Ws1_w[0], Ws1_w[1], Wc1_w[0], Wc1_w[1]],
                          axis=1).astype(jnp.bfloat16)

    grid = (b // _BT,)
    full = lambda a: pl.BlockSpec(a.shape, lambda i: (0,) * a.ndim)
    out = pl.pallas_call(
        _attn_kernel,
        grid=grid,
        in_specs=[
            pl.BlockSpec((_BT, _L, _D), lambda i: (i, 0, 0)),
            pl.BlockSpec((_N, 1), lambda i: (i, 0)),
            pl.BlockSpec((_N, 2), lambda i: (i, 0)),
            full(w1c), full(Ws1_b), full(Ws2_w), full(Ws2_b),
            full(Wc1_b), full(Wc2_w), full(Wc2_b),
        ],
        out_specs=pl.BlockSpec((_BT, _L, _D), lambda i: (i, 0, 0)),
        out_shape=jax.ShapeDtypeStruct((b, _L, _D), jnp.float32),
    )(element, frn, gboth, w1c, Ws1_b, Ws2_w, Ws2_b, Wc1_b, Wc2_w, Wc2_b)
    return out


# speedup vs baseline: 12.4443x; 1.1553x over previous
import jax
import jax.numpy as jnp
from jax import lax
from jax.experimental import pallas as pl

_B, _L, _D, _H, _HID = 256, 8, 128, 2, 256
_BT = 32
_N = _BT * _L * _L
_R = _BT * _L


def _leaky(x):
    return jnp.maximum(x, 0.01 * x)


def _attn_kernel(el_ref, pk_ref,
                 w1c_ref, ws1b_ref, ws2_ref, ws2b_ref,
                 wc1b_ref, wc2_ref, wc2b_ref, out_ref):
    el = el_ref[...]
    el2 = el.reshape(_R, _D)

    pk = pk_ref[...].reshape(3, _N)
    pkt = jnp.transpose(pk, (1, 0))
    gie = pkt[:, 0:1]
    gio = pkt[:, 1:2]
    fr = lax.bitcast_convert_type(pkt[:, 2:3], jnp.float32)

    base = pl.program_id(0) * _R
    col = lax.broadcasted_iota(jnp.int32, (_N, _R), 1) + base
    ohe = (gie == col).astype(jnp.bfloat16)
    oho = (gio == col).astype(jnp.bfloat16)
    el2b = el2.astype(jnp.bfloat16)
    ae = jnp.dot(ohe, el2b, preferred_element_type=jnp.float32)
    ao = jnp.dot(oho, el2b, preferred_element_type=jnp.float32)
    aeb = ae.astype(jnp.bfloat16)
    aob = ao.astype(jnp.bfloat16)

    halp = (jnp.dot(aeb, w1c_ref[:_D, :], preferred_element_type=jnp.float32)
            + jnp.dot(aob, w1c_ref[_D:, :], preferred_element_type=jnp.float32))

    acc = jnp.zeros((_R, _D), jnp.float32)
    for h in range(_H):
        h1 = _leaky(halp[:, h * _HID:(h + 1) * _HID]
                    + ws1b_ref[h][None, :]).astype(jnp.bfloat16)
        aw = (jnp.dot(h1, ws2_ref[h].astype(jnp.bfloat16),
                      preferred_element_type=jnp.float32)
              + ws2b_ref[h][None, :])
        w = jnp.exp(aw) * fr

        c1 = _leaky(halp[:, (2 + h) * _HID:(3 + h) * _HID]
                    + wc1b_ref[h][None, :]).astype(jnp.bfloat16)
        c = (jnp.dot(c1, wc2_ref[h].astype(jnp.bfloat16),
                     preferred_element_type=jnp.float32)
             + wc2b_ref[h][None, :])

        cw = c * w
        wl = w * jnp.ones((1, _D), jnp.float32)
        num = jnp.sum(cw.reshape(_R, _L, _D), axis=1)
        den = jnp.sum(wl.reshape(_R, _L, _D), axis=1)
        acc = acc + num / den

    res = el2 + acc * (1.0 / _H)
    out_ref[...] = res.reshape(_BT, _L, _D)


def kernel(element, fraction, element_indices, Ws1_w, Ws1_b, Ws2_w, Ws2_b,
           Wc1_w, Wc1_b, Wc2_w, Wc2_b):
    b = element.shape[0]
    ntiles = b // _BT
    boff = (_L * jnp.arange(b, dtype=jnp.int32))[:, None]
    gidx = element_indices.astype(jnp.int32) + boff
    frbits = lax.bitcast_convert_type(jnp.tile(fraction, (1, _L)), jnp.int32)
    packed = jnp.stack([gidx[:, 0::2].reshape(ntiles, _N),
                        gidx[:, 1::2].reshape(ntiles, _N),
                        frbits.reshape(ntiles, _N)], axis=1)
    w1c = jnp.concatenate([Ws1_w[0], Ws1_w[1], Wc1_w[0], Wc1_w[1]],
                          axis=1).astype(jnp.bfloat16)

    full = lambda a: pl.BlockSpec(a.shape, lambda i: (0,) * a.ndim)
    out = pl.pallas_call(
        _attn_kernel,
        grid=(ntiles,),
        in_specs=[
            pl.BlockSpec((_BT, _L, _D), lambda i: (i, 0, 0)),
            pl.BlockSpec((1, 3, _N), lambda i: (i, 0, 0)),
            full(w1c), full(Ws1_b), full(Ws2_w), full(Ws2_b),
            full(Wc1_b), full(Wc2_w), full(Wc2_b),
        ],
        out_specs=pl.BlockSpec((_BT, _L, _D), lambda i: (i, 0, 0)),
        out_shape=jax.ShapeDtypeStruct((b, _L, _D), jnp.float32),
    )(element, packed, w1c, Ws1_b, Ws2_w, Ws2_b, Wc1_b, Wc2_w, Wc2_b)
    return out
